# Initial kernel scaffold; baseline (speedup 1.0000x reference)
import jax
import jax.numpy as jnp
from jax.experimental import pallas as pl


def kernel(z, codebook):
    raise NotImplementedError("write your pallas kernel here")


# speedup vs baseline: 2.4921x; 2.4921x over previous
import jax
import jax.numpy as jnp
from jax.experimental import pallas as pl

_B = 512
_N = 1024
_D = 64
_COMMIT = 0.25


def _vq_body(x_ref, zsq_ref, csq_ref, cb_ref, qst_ref, idx_ref, loss_ref):
    i = pl.program_id(0)
    x = x_ref[...]
    cb = cb_ref[...]
    zc = jax.lax.dot_general(x, cb, (((1,), (1,)), ((), ())),
                             preferred_element_type=jnp.float32)
    dist = (zsq_ref[...] + csq_ref[...]) - 2.0 * zc
    dmin = jnp.min(dist, axis=1, keepdims=True)
    iota = jax.lax.broadcasted_iota(jnp.int32, (_B, _N), 1)
    idx = jnp.min(jnp.where(dist == dmin, iota, _N), axis=1)
    oh = (iota == idx[:, None]).astype(jnp.float32)
    q = jnp.dot(oh, cb, preferred_element_type=jnp.float32)
    d = q - x
    qst_ref[...] = x + d
    idx_ref[...] = idx
    part = jnp.sum(jnp.sum(d * d, axis=1, keepdims=True), axis=0,
                   keepdims=True)

    @pl.when(i == 0)
    def _():
        loss_ref[...] = jnp.zeros_like(loss_ref)

    loss_ref[...] += part


def kernel(z, codebook):
    b, dim, t = z.shape
    zf = jnp.transpose(z, (0, 2, 1)).reshape(-1, dim)
    zsq = jnp.sum(zf ** 2, axis=1, keepdims=True)
    csq = jnp.sum(codebook ** 2, axis=1).reshape(1, _N)
    n = zf.shape[0]
    qst, idx, lacc = pl.pallas_call(
        _vq_body,
        grid=(n // _B,),
        in_specs=[
            pl.BlockSpec((_B, dim), lambda i: (i, 0)),
            pl.BlockSpec((_B, 1), lambda i: (i, 0)),
            pl.BlockSpec((1, _N), lambda i: (0, 0)),
            pl.BlockSpec((_N, dim), lambda i: (0, 0)),
        ],
        out_specs=[
            pl.BlockSpec((_B, dim), lambda i: (i, 0)),
            pl.BlockSpec((_B,), lambda i: (i,)),
            pl.BlockSpec((1, 1), lambda i: (0, 0)),
        ],
        out_shape=[
            jax.ShapeDtypeStruct((n, dim), jnp.float32),
            jax.ShapeDtypeStruct((n,), jnp.int32),
            jax.ShapeDtypeStruct((1, 1), jnp.float32),
        ],
    )(zf, zsq, csq, codebook)
    quantized_st = jnp.transpose(qst.reshape(b, t, dim), (0, 2, 1))
    ls = lacc[0, 0] / (b * dim * t)
    loss = ls + _COMMIT * ls
    return (quantized_st, loss, idx[:, None])


# speedup vs baseline: 2.9497x; 1.1836x over previous
import jax
import jax.numpy as jnp
from jax.experimental import pallas as pl

_B = 512
_N = 1024
_D = 64
_L = 128
_COMMIT = 0.25


def _vq_body(z_ref, zsq_ref, csq_ref, cb_ref, qst_ref, idx_ref, loss_ref):
    i = pl.program_id(0)
    x = jnp.transpose(z_ref[0])
    cb = cb_ref[...]
    zc = jax.lax.dot_general(x, cb, (((1,), (1,)), ((), ())),
                             preferred_element_type=jnp.float32)
    w = zsq_ref[...]
    lane = jax.lax.broadcasted_iota(jnp.int32, (_B, _L), 1)
    v = ix = None
    for k in range(_N // _L):
        sl = slice(k * _L, (k + 1) * _L)
        dk = (w + csq_ref[:, sl]) - 2.0 * zc[:, sl]
        ik = lane + (k * _L)
        if v is None:
            v, ix = dk, ik
        else:
            m = v <= dk
            v = jnp.where(m, v, dk)
            ix = jnp.where(m, ix, ik)
    dmin = jnp.min(v, axis=1, keepdims=True)
    idx = jnp.min(jnp.where(v == dmin, ix, _N), axis=1)
    oh = (jax.lax.broadcasted_iota(jnp.int32, (_B, _N), 1)
          == idx[:, None]).astype(jnp.float32)
    q = jnp.dot(oh, cb, preferred_element_type=jnp.float32)
    d = q - x
    qst_ref[0] = jnp.transpose(x + d)
    idx_ref[...] = idx
    part = jnp.sum(jnp.sum(d * d, axis=1, keepdims=True), axis=0,
                   keepdims=True)

    @pl.when(i == 0)
    def _():
        loss_ref[...] = jnp.zeros_like(loss_ref)

    loss_ref[...] += part


def kernel(z, codebook):
    b, dim, t = z.shape
    zf = jnp.transpose(z, (0, 2, 1)).reshape(-1, dim)
    zsq = jnp.sum(zf ** 2, axis=1, keepdims=True)
    csq = jnp.sum(codebook ** 2, axis=1).reshape(1, _N)
    n = b * t
    tpb = t // _B
    qst, idx, lacc = pl.pallas_call(
        _vq_body,
        grid=(n // _B,),
        in_specs=[
            pl.BlockSpec((1, dim, _B), lambda i: (i // tpb, 0, i % tpb)),
            pl.BlockSpec((_B, 1), lambda i: (i, 0)),
            pl.BlockSpec((1, _N), lambda i: (0, 0)),
            pl.BlockSpec((_N, dim), lambda i: (0, 0)),
        ],
        out_specs=[
            pl.BlockSpec((1, dim, _B), lambda i: (i // tpb, 0, i % tpb)),
            pl.BlockSpec((_B,), lambda i: (i,)),
            pl.BlockSpec((1, 1), lambda i: (0, 0)),
        ],
        out_shape=[
            jax.ShapeDtypeStruct((b, dim, t), jnp.float32),
            jax.ShapeDtypeStruct((n,), jnp.int32),
            jax.ShapeDtypeStruct((1, 1), jnp.float32),
        ],
    )(z, zsq, csq, codebook)
    ls = lacc[0, 0] / (b * dim * t)
    loss = ls + _COMMIT * ls
    return (qst, loss, idx[:, None])


# speedup vs baseline: 4.6419x; 1.5737x over previous
import jax
import jax.numpy as jnp
from jax.experimental import pallas as pl

_B = 512
_N = 1024
_D = 64
_S = 8
_COMMIT = 0.25


def _vq_body(z_ref, zsq_ref, csq_ref, cb_ref, cbt_ref, qst_ref, idx_ref,
             loss_ref):
    i = pl.program_id(0)
    xt = z_ref[0]
    cb = cb_ref[...]
    zc = jnp.dot(cb, xt, preferred_element_type=jnp.float32)
    w = zsq_ref[0]
    si = jax.lax.broadcasted_iota(jnp.int32, (_S, _B), 0)
    v = ix = None
    for k in range(_N // _S):
        sl = slice(k * _S, (k + 1) * _S)
        dk = (w + csq_ref[sl, :]) - 2.0 * zc[sl, :]
        ik = si + (k * _S)
        if v is None:
            v, ix = dk, ik
        else:
            m = v <= dk
            v = jnp.where(m, v, dk)
            ix = jnp.where(m, ix, ik)
    dmin = jnp.min(v, axis=0, keepdims=True)
    idx = jnp.min(jnp.where(v == dmin, ix, _N), axis=0, keepdims=True)
    oh = (jax.lax.broadcasted_iota(jnp.int32, (_N, _B), 0)
          == idx).astype(jnp.float32)
    q = jnp.dot(cbt_ref[...], oh, preferred_element_type=jnp.float32)
    d = q - xt
    qst_ref[0] = xt + d
    idx_ref[0] = idx
    part = jnp.sum(jnp.sum(d * d, axis=1, keepdims=True), axis=0,
                   keepdims=True)

    @pl.when(i == 0)
    def _():
        loss_ref[...] = jnp.zeros_like(loss_ref)

    loss_ref[...] += part


def kernel(z, codebook):
    b, dim, t = z.shape
    zf = jnp.transpose(z, (0, 2, 1)).reshape(-1, dim)
    zsq = jnp.sum(zf ** 2, axis=1, keepdims=True)
    zsq3 = zsq.reshape(b, 1, t)
    csq = jnp.sum(codebook ** 2, axis=1).reshape(_N, 1)
    cbt = jnp.transpose(codebook)
    n = b * t
    tpb = t // _B
    qst, idx, lacc = pl.pallas_call(
        _vq_body,
        grid=(n // _B,),
        in_specs=[
            pl.BlockSpec((1, dim, _B), lambda i: (i // tpb, 0, i % tpb)),
            pl.BlockSpec((1, 1, _B), lambda i: (i // tpb, 0, i % tpb)),
            pl.BlockSpec((_N, 1), lambda i: (0, 0)),
            pl.BlockSpec((_N, dim), lambda i: (0, 0)),
            pl.BlockSpec((dim, _N), lambda i: (0, 0)),
        ],
        out_specs=[
            pl.BlockSpec((1, dim, _B), lambda i: (i // tpb, 0, i % tpb)),
            pl.BlockSpec((1, 1, _B), lambda i: (i // tpb, 0, i % tpb)),
            pl.BlockSpec((1, 1), lambda i: (0, 0)),
        ],
        out_shape=[
            jax.ShapeDtypeStruct((b, dim, t), jnp.float32),
            jax.ShapeDtypeStruct((b, 1, t), jnp.int32),
            jax.ShapeDtypeStruct((1, 1), jnp.float32),
        ],
    )(z, zsq3, csq, codebook, cbt)
    ls = lacc[0, 0] / (b * dim * t)
    loss = ls + _COMMIT * ls
    return (qst, loss, idx.reshape(n, 1))


# speedup vs baseline: 6.0146x; 1.2957x over previous
import jax
import jax.numpy as jnp
from jax.experimental import pallas as pl

_B = 1024
_N = 1024
_D = 64
_S = 8
_COMMIT = 0.25


def _vq_body(z_ref, zsq_ref, csq_ref, cbm2_ref, cbt_ref, qst_ref, idx_ref,
             loss_ref):
    i = pl.program_id(0)
    xt = z_ref[0]
    cbm2 = cbm2_ref[...]
    zc2 = jnp.dot(cbm2, xt, preferred_element_type=jnp.float32)
    w = zsq_ref[0]
    wb = jnp.broadcast_to(w, (_S, _B))
    si = jax.lax.broadcasted_iota(jnp.int32, (_S, _B), 0)
    v = ix = None
    for k in range(_N // _S):
        sl = slice(k * _S, (k + 1) * _S)
        dk = (wb + csq_ref[sl, :]) + zc2[sl, :]
        ik = si + (k * _S)
        if v is None:
            v, ix = dk, ik
        else:
            m = v <= dk
            v = jnp.where(m, v, dk)
            ix = jnp.where(m, ix, ik)
    dmin = jnp.min(v, axis=0, keepdims=True)
    idx = jnp.min(jnp.where(v == dmin, ix, _N), axis=0, keepdims=True)
    oh = (jax.lax.broadcasted_iota(jnp.int32, (_N, _B), 0)
          == idx).astype(jnp.float32)
    q = jnp.dot(cbt_ref[...], oh, preferred_element_type=jnp.float32)
    d = q - xt
    qst_ref[0] = xt + d
    idx_ref[0] = idx
    part = jnp.sum(jnp.sum(d * d, axis=1, keepdims=True), axis=0,
                   keepdims=True)

    @pl.when(i == 0)
    def _():
        loss_ref[...] = jnp.zeros_like(loss_ref)

    loss_ref[...] += part


def kernel(z, codebook):
    b, dim, t = z.shape
    zf = jnp.transpose(z, (0, 2, 1)).reshape(-1, dim)
    zsq = jnp.sum(zf ** 2, axis=1, keepdims=True)
    zsq3 = zsq.reshape(b, 1, t)
    csq = jnp.sum(codebook ** 2, axis=1).reshape(_N, 1)
    cbm2 = -2.0 * codebook
    cbt = jnp.transpose(codebook)
    n = b * t
    tpb = t // _B
    qst, idx, lacc = pl.pallas_call(
        _vq_body,
        grid=(n // _B,),
        in_specs=[
            pl.BlockSpec((1, dim, _B), lambda i: (i // tpb, 0, i % tpb)),
            pl.BlockSpec((1, 1, _B), lambda i: (i // tpb, 0, i % tpb)),
            pl.BlockSpec((_N, 1), lambda i: (0, 0)),
            pl.BlockSpec((_N, dim), lambda i: (0, 0)),
            pl.BlockSpec((dim, _N), lambda i: (0, 0)),
        ],
        out_specs=[
            pl.BlockSpec((1, dim, _B), lambda i: (i // tpb, 0, i % tpb)),
            pl.BlockSpec((1, 1, _B), lambda i: (i // tpb, 0, i % tpb)),
            pl.BlockSpec((1, 1), lambda i: (0, 0)),
        ],
        out_shape=[
            jax.ShapeDtypeStruct((b, dim, t), jnp.float32),
            jax.ShapeDtypeStruct((b, 1, t), jnp.int32),
            jax.ShapeDtypeStruct((1, 1), jnp.float32),
        ],
    )(z, zsq3, csq, cbm2, cbt)
    ls = lacc[0, 0] / (b * dim * t)
    loss = ls + _COMMIT * ls
    return (qst, loss, idx.reshape(n, 1))


# speedup vs baseline: 6.6494x; 1.1055x over previous
import jax
import jax.numpy as jnp
from jax.experimental import pallas as pl

_B = 1024
_N = 1024
_D = 64
_S = 8
_COMMIT = 0.25


def _vq_body(z_ref, csq_ref, cbm2_ref, cbt_ref, qst_ref, idx_ref,
             loss_ref):
    i = pl.program_id(0)
    xt = z_ref[0]
    cbm2 = cbm2_ref[...]
    zc2 = jnp.dot(cbm2, xt, preferred_element_type=jnp.float32)
    w = jnp.sum(xt * xt, axis=0, keepdims=True)
    wb = jnp.broadcast_to(w, (_S, _B))
    si = jax.lax.broadcasted_iota(jnp.int32, (_S, _B), 0)
    v = ix = None
    for k in range(_N // _S):
        sl = slice(k * _S, (k + 1) * _S)
        dk = (wb + csq_ref[sl, :]) + zc2[sl, :]
        ik = si + (k * _S)
        if v is None:
            v, ix = dk, ik
        else:
            m = v <= dk
            v = jnp.where(m, v, dk)
            ix = jnp.where(m, ix, ik)
    dmin = jnp.min(v, axis=0, keepdims=True)
    idx = jnp.min(jnp.where(v == dmin, ix, _N), axis=0, keepdims=True)
    oh = (jax.lax.broadcasted_iota(jnp.int32, (_N, _B), 0)
          == idx).astype(jnp.float32)
    q = jnp.dot(cbt_ref[...], oh, preferred_element_type=jnp.float32)
    d = q - xt
    qst_ref[0] = xt + d
    idx_ref[0] = idx
    part = jnp.sum(jnp.sum(d * d, axis=1, keepdims=True), axis=0,
                   keepdims=True)

    @pl.when(i == 0)
    def _():
        loss_ref[...] = jnp.zeros_like(loss_ref)

    loss_ref[...] += part


def kernel(z, codebook):
    b, dim, t = z.shape
    csq = jnp.sum(codebook ** 2, axis=1).reshape(_N, 1)
    cbm2 = -2.0 * codebook
    cbt = jnp.transpose(codebook)
    n = b * t
    tpb = t // _B
    qst, idx, lacc = pl.pallas_call(
        _vq_body,
        grid=(n // _B,),
        in_specs=[
            pl.BlockSpec((1, dim, _B), lambda i: (i // tpb, 0, i % tpb)),
            pl.BlockSpec((_N, 1), lambda i: (0, 0)),
            pl.BlockSpec((_N, dim), lambda i: (0, 0)),
            pl.BlockSpec((dim, _N), lambda i: (0, 0)),
        ],
        out_specs=[
            pl.BlockSpec((1, dim, _B), lambda i: (i // tpb, 0, i % tpb)),
            pl.BlockSpec((1, 1, _B), lambda i: (i // tpb, 0, i % tpb)),
            pl.BlockSpec((1, 1), lambda i: (0, 0)),
        ],
        out_shape=[
            jax.ShapeDtypeStruct((b, dim, t), jnp.float32),
            jax.ShapeDtypeStruct((b, 1, t), jnp.int32),
            jax.ShapeDtypeStruct((1, 1), jnp.float32),
        ],
    )(z, csq, cbm2, cbt)
    ls = lacc[0, 0] / (b * dim * t)
    loss = ls + _COMMIT * ls
    return (qst, loss, idx.reshape(n, 1))


# speedup vs baseline: 7.0366x; 1.0582x over previous
import jax
import jax.numpy as jnp
from jax.experimental import pallas as pl

_B = 1024
_N = 1024
_D = 64
_S = 8
_COMMIT = 0.25


def _vq_body(z_ref, cb_ref, qst_ref, idx_ref, loss_ref):
    i = pl.program_id(0)
    xt = z_ref[0]
    cb = cb_ref[...]
    cbm2 = -2.0 * cb
    csq = jnp.sum(cb * cb, axis=1, keepdims=True)
    zc2 = jnp.dot(cbm2, xt, preferred_element_type=jnp.float32)
    w = jnp.sum(xt * xt, axis=0, keepdims=True)
    wb = jnp.broadcast_to(w, (_S, _B))
    si = jax.lax.broadcasted_iota(jnp.int32, (_S, _B), 0)
    items = []
    for k in range(_N // _S):
        sl = slice(k * _S, (k + 1) * _S)
        dk = (wb + csq[sl, :]) + zc2[sl, :]
        items.append((dk, si + (k * _S)))
    while len(items) > 1:
        nxt = []
        for j in range(0, len(items) - 1, 2):
            va, ia = items[j]
            vb, ib = items[j + 1]
            m = va <= vb
            nxt.append((jnp.where(m, va, vb), jnp.where(m, ia, ib)))
        if len(items) % 2:
            nxt.append(items[-1])
        items = nxt
    v, ix = items[0]
    dmin = jnp.min(v, axis=0, keepdims=True)
    idx = jnp.min(jnp.where(v == dmin, ix, _N), axis=0, keepdims=True)
    oh = (jax.lax.broadcasted_iota(jnp.int32, (_N, _B), 0)
          == idx).astype(jnp.float32)
    q = jnp.dot(jnp.transpose(cb), oh,
                preferred_element_type=jnp.float32)
    d = q - xt
    qst_ref[0] = xt + d
    idx_ref[0] = idx
    part = jnp.sum(jnp.sum(d * d, axis=1, keepdims=True), axis=0,
                   keepdims=True)

    @pl.when(i == 0)
    def _():
        loss_ref[...] = jnp.zeros_like(loss_ref)

    loss_ref[...] += part


def kernel(z, codebook):
    b, dim, t = z.shape
    n = b * t
    tpb = t // _B
    qst, idx, lacc = pl.pallas_call(
        _vq_body,
        grid=(n // _B,),
        in_specs=[
            pl.BlockSpec((1, dim, _B), lambda i: (i // tpb, 0, i % tpb)),
            pl.BlockSpec((_N, dim), lambda i: (0, 0)),
        ],
        out_specs=[
            pl.BlockSpec((1, dim, _B), lambda i: (i // tpb, 0, i % tpb)),
            pl.BlockSpec((1, 1, _B), lambda i: (i // tpb, 0, i % tpb)),
            pl.BlockSpec((1, 1), lambda i: (0, 0)),
        ],
        out_shape=[
            jax.ShapeDtypeStruct((b, dim, t), jnp.float32),
            jax.ShapeDtypeStruct((b, 1, t), jnp.int32),
            jax.ShapeDtypeStruct((1, 1), jnp.float32),
        ],
    )(z, codebook)
    ls = lacc[0, 0] / (b * dim * t)
    loss = ls + _COMMIT * ls
    return (qst, loss, idx.reshape(n, 1))


# speedup vs baseline: 7.1758x; 1.0198x over previous
import jax
import jax.numpy as jnp
from jax.experimental import pallas as pl

_B = 1024
_N = 1024
_D = 64
_S = 8
_COMMIT = 0.25


def _vq_body(z_ref, cb_ref, qst_ref, idx_ref, loss_ref):
    i = pl.program_id(0)
    xt = z_ref[0]
    cb = cb_ref[...]
    cbm2 = -2.0 * cb
    csq = jnp.sum(cb * cb, axis=1, keepdims=True)
    zc2 = jnp.dot(cbm2, xt, preferred_element_type=jnp.float32)
    w = jnp.sum(xt * xt, axis=0, keepdims=True)
    wb = jnp.broadcast_to(w, (_S, _B))
    si = jax.lax.broadcasted_iota(jnp.int32, (_S, _B), 0)
    ch = 4
    per = (_N // _S) // ch
    items = []
    for c in range(ch):
        v = ixv = None
        for k in range(c * per, (c + 1) * per):
            sl = slice(k * _S, (k + 1) * _S)
            dk = (wb + csq[sl, :]) + zc2[sl, :]
            ik = si + (k * _S)
            if v is None:
                v, ixv = dk, ik
            else:
                m = v <= dk
                v = jnp.where(m, v, dk)
                ixv = jnp.where(m, ixv, ik)
        items.append((v, ixv))
    while len(items) > 1:
        nxt = []
        for j in range(0, len(items), 2):
            va, ia = items[j]
            vb, ib = items[j + 1]
            m = va <= vb
            nxt.append((jnp.where(m, va, vb), jnp.where(m, ia, ib)))
        items = nxt
    v, ix = items[0]
    dmin = jnp.min(v, axis=0, keepdims=True)
    idx = jnp.min(jnp.where(v == dmin, ix, _N), axis=0, keepdims=True)
    oh = (jax.lax.broadcasted_iota(jnp.int32, (_N, _B), 0)
          == idx).astype(jnp.float32)
    q = jnp.dot(jnp.transpose(cb), oh,
                preferred_element_type=jnp.float32)
    d = q - xt
    qst_ref[0] = xt + d
    idx_ref[0] = idx
    part = jnp.sum(jnp.sum(d * d, axis=1, keepdims=True), axis=0,
                   keepdims=True)

    @pl.when(i == 0)
    def _():
        loss_ref[...] = jnp.zeros_like(loss_ref)

    loss_ref[...] += part


def kernel(z, codebook):
    b, dim, t = z.shape
    n = b * t
    tpb = t // _B
    qst, idx, lacc = pl.pallas_call(
        _vq_body,
        grid=(n // _B,),
        in_specs=[
            pl.BlockSpec((1, dim, _B), lambda i: (i // tpb, 0, i % tpb)),
            pl.BlockSpec((_N, dim), lambda i: (0, 0)),
        ],
        out_specs=[
            pl.BlockSpec((1, dim, _B), lambda i: (i // tpb, 0, i % tpb)),
            pl.BlockSpec((1, 1, _B), lambda i: (i // tpb, 0, i % tpb)),
            pl.BlockSpec((1, 1), lambda i: (0, 0)),
        ],
        out_shape=[
            jax.ShapeDtypeStruct((b, dim, t), jnp.float32),
            jax.ShapeDtypeStruct((b, 1, t), jnp.int32),
            jax.ShapeDtypeStruct((1, 1), jnp.float32),
        ],
    )(z, codebook)
    ls = lacc[0, 0] / (b * dim * t)
    loss = ls + _COMMIT * ls
    return (qst, loss, idx.reshape(n, 1))
